# baseline (speedup 1.0000x reference)
import jax
import jax.numpy as jnp
from jax.experimental import pallas as pl
from jax.experimental.pallas import tpu as pltpu


_TILE = 256
_NBUF = 3


def _body(x_ref, emb_ref, out_ref,
          xb0, xb1, xb2, eb0, eb1, eb2,
          xs0, xs1, xs2, es0, es1, es2, os0, os1, os2):
    xbufs = (xb0, xb1, xb2)
    ebufs = (eb0, eb1, eb2)
    xsems = (xs0, xs1, xs2)
    esems = (es0, es1, es2)
    osems = (os0, os1, os2)
    n_tiles = x_ref.shape[1] // _TILE

    def load(t):
        s = t % _NBUF
        lx = pltpu.make_async_copy(
            x_ref.at[:, pl.ds(t * _TILE, _TILE), :], xbufs[s], xsems[s])
        le = pltpu.make_async_copy(
            emb_ref.at[pl.ds(t * _TILE, _TILE), :], ebufs[s], esems[s])
        lx.start()
        le.start()
        return lx, le

    def store(t):
        s = t % _NBUF
        st = pltpu.make_async_copy(
            xbufs[s], out_ref.at[:, pl.ds(t * _TILE, _TILE), :], osems[s])
        st.start()
        return st

    lds = [None] * n_tiles
    sts = [None] * n_tiles
    for t in range(min(_NBUF, n_tiles)):
        lds[t] = load(t)
    for t in range(n_tiles):
        s = t % _NBUF
        if t >= 1 and t + 2 < n_tiles:
            sts[t - 1].wait()
            lds[t + 2] = load(t + 2)
        lds[t][0].wait()
        lds[t][1].wait()
        xbufs[s][...] = xbufs[s][...] + ebufs[s][...][None, :, :]
        sts[t] = store(t)
    for t in range(max(0, n_tiles - _NBUF), n_tiles):
        sts[t].wait()


def kernel(x, emb_weight):
    bs, seq_len, ch = x.shape
    emb = emb_weight[:seq_len]
    return pl.pallas_call(
        _body,
        in_specs=[
            pl.BlockSpec(memory_space=pl.ANY),
            pl.BlockSpec(memory_space=pl.ANY),
        ],
        out_specs=pl.BlockSpec(memory_space=pl.ANY),
        out_shape=jax.ShapeDtypeStruct((bs, seq_len, ch), x.dtype),
        scratch_shapes=(
            [pltpu.VMEM((bs, _TILE, ch), x.dtype) for _ in range(_NBUF)]
            + [pltpu.VMEM((_TILE, ch), x.dtype) for _ in range(_NBUF)]
            + [pltpu.SemaphoreType.DMA] * (3 * _NBUF)
        ),
    )(x, emb)


# speedup vs baseline: 1.0142x; 1.0142x over previous
import jax
import jax.numpy as jnp
from jax.experimental import pallas as pl


_SEQ_BLOCK = 512


def _add_kernel(x_ref, emb_ref, out_ref):
    out_ref[...] = x_ref[...] + emb_ref[...][None, :, :]


def kernel(x, emb_weight):
    bs, seq_len, ch = x.shape
    emb = emb_weight[:seq_len]
    blk = _SEQ_BLOCK if seq_len % _SEQ_BLOCK == 0 else seq_len
    grid = (seq_len // blk,)
    return pl.pallas_call(
        _add_kernel,
        grid=grid,
        in_specs=[
            pl.BlockSpec((bs, blk, ch), lambda i: (0, i, 0)),
            pl.BlockSpec((blk, ch), lambda i: (i, 0)),
        ],
        out_specs=pl.BlockSpec((bs, blk, ch), lambda i: (0, i, 0)),
        out_shape=jax.ShapeDtypeStruct((bs, seq_len, ch), x.dtype),
    )(x, emb)
